# baseline (speedup 1.0000x reference)
import functools

import jax
import jax.numpy as jnp
from jax import lax
from jax.experimental import pallas as pl
from jax.experimental.pallas import tpu as pltpu
from jax.experimental.pallas import tpu_sc as plsc

_E = 8
_TOPK = 2
_D = 2048
_DFF = 8192
_T = 4096
_N = _T * _TOPK
_BLK = 256
_NBLK = _N // _BLK + _E
_NPAD = _NBLK * _BLK
_DFFT = 1024
_NJ = _DFF // _DFFT
_NC = 2
_NS = 16
_NW = _NC * _NS
_GCH = 32
_CT = 512


def _gate_body(x_ref, wg_ref, o_ref):
    logits = lax.dot_general(
        x_ref[...], wg_ref[...], (((1,), (1,)), ((), ())),
        preferred_element_type=jnp.float32)
    o_ref[...] = jax.nn.softmax(logits, axis=0)


def _gate(xf, wg):
    return pl.pallas_call(
        _gate_body,
        out_shape=jax.ShapeDtypeStruct((_T, _E), jnp.float32),
    )(xf, wg)


def _ffn_body(bexp_ref, nu_ref, x_ref, acc_ref, w1_ref, b1_ref, w2_ref,
              b2_ref, o_ref, w1b_ref, w2b_ref):
    j = pl.program_id(0)
    i = pl.program_id(1)
    cur = bexp_ref[i]
    prev = bexp_ref[jnp.maximum(i - 1, 0)]

    @pl.when((i == 0) | (cur != prev))
    def _():
        w1b_ref[...] = w1_ref[0].astype(jnp.bfloat16)
        w2b_ref[...] = w2_ref[0].astype(jnp.bfloat16)

    @pl.when(i < nu_ref[0])
    def _():
        h = jnp.dot(x_ref[...], w1b_ref[...],
                    preferred_element_type=jnp.float32)
        h = jax.nn.gelu(h + b1_ref[0])
        part = jnp.dot(h.astype(jnp.bfloat16), w2b_ref[...],
                       preferred_element_type=jnp.float32)

        @pl.when(j == 0)
        def _():
            o_ref[...] = part + b2_ref[0]

        @pl.when(j > 0)
        def _():
            o_ref[...] = acc_ref[...] + part


def _ffn(bexp, nused, xs, W1, b1, W2, b2):
    grid_spec = pltpu.PrefetchScalarGridSpec(
        num_scalar_prefetch=2,
        grid=(_NJ, _NBLK),
        in_specs=[
            pl.BlockSpec((_BLK, _D), lambda j, i, be, nu: (i, 0)),
            pl.BlockSpec((_BLK, _D), lambda j, i, be, nu: (i, 0)),
            pl.BlockSpec((1, _D, _DFFT), lambda j, i, be, nu: (be[i], 0, j)),
            pl.BlockSpec((1, 1, _DFFT), lambda j, i, be, nu: (be[i], 0, j)),
            pl.BlockSpec((1, _DFFT, _D), lambda j, i, be, nu: (be[i], j, 0)),
            pl.BlockSpec((1, 1, _D), lambda j, i, be, nu: (be[i], 0, 0)),
        ],
        out_specs=pl.BlockSpec((_BLK, _D), lambda j, i, be, nu: (i, 0)),
        scratch_shapes=[
            pltpu.VMEM((_D, _DFFT), jnp.bfloat16),
            pltpu.VMEM((_DFFT, _D), jnp.bfloat16),
        ],
    )
    acc = jnp.zeros((_NPAD, _D), jnp.float32)
    return pl.pallas_call(
        _ffn_body,
        grid_spec=grid_spec,
        out_shape=jax.ShapeDtypeStruct((_NPAD, _D), jnp.float32),
        input_output_aliases={3: 0},
        compiler_params=pltpu.CompilerParams(
            dimension_semantics=("arbitrary", "arbitrary"),
            vmem_limit_bytes=60 * 1024 * 1024),
    )(bexp, nused, xs, acc, W1, b1.reshape(_E, 1, _DFF), W2,
      b2.reshape(_E, 1, _D))


@functools.lru_cache(maxsize=None)
def _make_sc_gather(n_rows, width):
    per_w = n_rows // _NW
    nch = per_w // _GCH
    mesh = plsc.VectorSubcoreMesh(
        core_axis_name="c", subcore_axis_name="s",
        num_cores=_NC, num_subcores=_NS)

    @functools.partial(
        pl.kernel,
        out_type=jax.ShapeDtypeStruct((n_rows, width), jnp.float32),
        mesh=mesh,
        scratch_types=[
            pltpu.VMEM((_GCH,), jnp.int32),
            pltpu.VMEM((_GCH, width), jnp.float32),
            pltpu.SemaphoreType.DMA,
        ],
    )
    def gather_k(table_hbm, idx_hbm, out_hbm, idx_v, rows_v, sem):
        wid = lax.axis_index("s") * _NC + lax.axis_index("c")
        base = wid * per_w

        def body(i, carry):
            off = base + i * _GCH
            pltpu.sync_copy(idx_hbm.at[pl.ds(off, _GCH)], idx_v)
            pltpu.async_copy(table_hbm.at[idx_v], rows_v, sem).wait()
            pltpu.sync_copy(rows_v, out_hbm.at[pl.ds(off, _GCH)])
            return carry

        lax.fori_loop(0, nch, body, 0)

    return gather_k


def _dispatch(xb_packed, gidx):
    return _make_sc_gather(_NPAD, _D // 2)(xb_packed, gidx)


def _combine_gather(y, slot):
    return _make_sc_gather(_N, _D)(y, slot)


def _comb_body(y0_ref, y1_ref, l_ref, m_ref, o_ref):
    eids = lax.broadcasted_iota(jnp.int32, (1, _E), 1)
    m = m_ref[...]
    l = l_ref[...]
    w0 = jnp.sum(jnp.where(m[:, 0:1] == eids, l, 0.0), axis=1, keepdims=True)
    w1 = jnp.sum(jnp.where(m[:, 1:2] == eids, l, 0.0), axis=1, keepdims=True)
    s = w0 + w1
    o_ref[...] = (y0_ref[...] * w0 + y1_ref[...] * w1) / s


def _combine(ys, logits, mapping):
    return pl.pallas_call(
        _comb_body,
        grid=(_T // _CT,),
        in_specs=[
            pl.BlockSpec((_CT, _D), lambda i: (i, 0)),
            pl.BlockSpec((_CT, _D), lambda i: (i + _T // _CT, 0)),
            pl.BlockSpec((_CT, _E), lambda i: (i, 0)),
            pl.BlockSpec((_CT, _TOPK), lambda i: (i, 0)),
        ],
        out_specs=pl.BlockSpec((_CT, _D), lambda i: (i, 0)),
        out_shape=jax.ShapeDtypeStruct((_T, _D), jnp.float32),
    )(ys, ys, logits, mapping)


def kernel(x, mapping, Wg, W1, b1, W2, b2):
    xf = x.reshape(_T, _D)
    logits = _gate(xf, Wg)

    mflat = jnp.concatenate([mapping[:, 0], mapping[:, 1]])
    oh = (mflat[:, None] == jnp.arange(_E, dtype=jnp.int32)[None, :])
    ranks = jnp.cumsum(oh.astype(jnp.int32), axis=0)
    rank = jnp.take_along_axis(ranks, mflat[:, None], axis=1)[:, 0] - 1
    counts = ranks[-1]
    padded = ((counts + _BLK - 1) // _BLK) * _BLK
    start = jnp.concatenate(
        [jnp.zeros((1,), padded.dtype), jnp.cumsum(padded)[:-1]])
    slot = (start[mflat] + rank).astype(jnp.int32)
    gidx = jnp.zeros((_NPAD,), jnp.int32).at[slot].set(
        jnp.arange(_N, dtype=jnp.int32) % _T)
    start_blk = (start // _BLK).astype(jnp.int32)
    bexp = (jnp.sum(
        jnp.arange(_NBLK, dtype=jnp.int32)[:, None] >= start_blk[None, :],
        axis=1) - 1).astype(jnp.int32)
    nused = ((jnp.sum(padded) // _BLK).astype(jnp.int32)).reshape(1)

    xb = lax.bitcast_convert_type(
        xf.astype(jnp.bfloat16).reshape(_T, _D // 2, 2), jnp.float32)
    xs = lax.bitcast_convert_type(
        _dispatch(xb, gidx), jnp.bfloat16).reshape(_NPAD, _D)
    y = _ffn(bexp, nused, xs, W1, b1, W2, b2)
    ys = _combine_gather(y, slot)
    return _combine(ys, logits, mapping)


# speedup vs baseline: 1.2795x; 1.2795x over previous
import functools

import jax
import jax.numpy as jnp
from jax import lax
from jax.experimental import pallas as pl
from jax.experimental.pallas import tpu as pltpu
from jax.experimental.pallas import tpu_sc as plsc

_E = 8
_TOPK = 2
_D = 2048
_DFF = 8192
_T = 4096
_N = _T * _TOPK
_BLK = 256
_NBLK = _N // _BLK + _E
_NPAD = _NBLK * _BLK
_DFFT = 1024
_NJ = _DFF // _DFFT
_NC = 2
_NS = 16
_NW = _NC * _NS
_GCH = 32
_CT = 512


def _gate_body(x_ref, wg_ref, o_ref):
    logits = lax.dot_general(
        x_ref[...], wg_ref[...], (((1,), (1,)), ((), ())),
        preferred_element_type=jnp.float32)
    o_ref[...] = jax.nn.softmax(logits, axis=0)


def _gate(xf, wg):
    return pl.pallas_call(
        _gate_body,
        out_shape=jax.ShapeDtypeStruct((_T, _E), jnp.float32),
    )(xf, wg)


def _ffn_body(bexp_ref, nu_ref, x_ref, acc_ref, w1_ref, b1_ref, w2_ref,
              b2_ref, o_ref, w1b_ref, w2b_ref):
    j = pl.program_id(0)
    i = pl.program_id(1)
    cur = bexp_ref[i]
    prev = bexp_ref[jnp.maximum(i - 1, 0)]

    @pl.when((i == 0) | (cur != prev))
    def _():
        w1b_ref[...] = w1_ref[0].astype(jnp.bfloat16)
        w2b_ref[...] = w2_ref[0].astype(jnp.bfloat16)

    @pl.when(i < nu_ref[0])
    def _():
        h = jnp.dot(x_ref[...].astype(jnp.bfloat16), w1b_ref[...],
                    preferred_element_type=jnp.float32)
        h = jax.nn.gelu(h + b1_ref[0])
        part = jnp.dot(h.astype(jnp.bfloat16), w2b_ref[...],
                       preferred_element_type=jnp.float32)

        @pl.when(j == 0)
        def _():
            o_ref[...] = part + b2_ref[0]

        @pl.when(j > 0)
        def _():
            o_ref[...] = acc_ref[...] + part


def _ffn(bexp, nused, xs, W1, b1, W2, b2):
    grid_spec = pltpu.PrefetchScalarGridSpec(
        num_scalar_prefetch=2,
        grid=(_NJ, _NBLK),
        in_specs=[
            pl.BlockSpec((_BLK, _D), lambda j, i, be, nu: (i, 0)),
            pl.BlockSpec((_BLK, _D), lambda j, i, be, nu: (i, 0)),
            pl.BlockSpec((1, _D, _DFFT), lambda j, i, be, nu: (be[i], 0, j)),
            pl.BlockSpec((1, 1, _DFFT), lambda j, i, be, nu: (be[i], 0, j)),
            pl.BlockSpec((1, _DFFT, _D), lambda j, i, be, nu: (be[i], j, 0)),
            pl.BlockSpec((1, 1, _D), lambda j, i, be, nu: (be[i], 0, 0)),
        ],
        out_specs=pl.BlockSpec((_BLK, _D), lambda j, i, be, nu: (i, 0)),
        scratch_shapes=[
            pltpu.VMEM((_D, _DFFT), jnp.bfloat16),
            pltpu.VMEM((_DFFT, _D), jnp.bfloat16),
        ],
    )
    acc = jnp.zeros((_NPAD, _D), jnp.float32)
    return pl.pallas_call(
        _ffn_body,
        grid_spec=grid_spec,
        out_shape=jax.ShapeDtypeStruct((_NPAD, _D), jnp.float32),
        input_output_aliases={3: 0},
        compiler_params=pltpu.CompilerParams(
            dimension_semantics=("arbitrary", "arbitrary"),
            vmem_limit_bytes=60 * 1024 * 1024),
    )(bexp, nused, xs, acc, W1, b1.reshape(_E, 1, _DFF), W2,
      b2.reshape(_E, 1, _D))


@functools.lru_cache(maxsize=None)
def _make_sc_gather(n_rows, width):
    per_w = n_rows // _NW
    nch = per_w // _GCH
    mesh = plsc.VectorSubcoreMesh(
        core_axis_name="c", subcore_axis_name="s",
        num_cores=_NC, num_subcores=_NS)

    @functools.partial(
        pl.kernel,
        out_type=jax.ShapeDtypeStruct((n_rows, width), jnp.float32),
        mesh=mesh,
        scratch_types=[
            pltpu.VMEM((_GCH,), jnp.int32),
            pltpu.VMEM((_GCH, width), jnp.float32),
            pltpu.SemaphoreType.DMA,
        ],
    )
    def gather_k(table_hbm, idx_hbm, out_hbm, idx_v, rows_v, sem):
        wid = lax.axis_index("s") * _NC + lax.axis_index("c")
        base = wid * per_w

        def body(i, carry):
            off = base + i * _GCH
            pltpu.sync_copy(idx_hbm.at[pl.ds(off, _GCH)], idx_v)
            pltpu.async_copy(table_hbm.at[idx_v], rows_v, sem).wait()
            pltpu.sync_copy(rows_v, out_hbm.at[pl.ds(off, _GCH)])
            return carry

        lax.fori_loop(0, nch, body, 0)

    return gather_k


def _dispatch(xf, gidx):
    return _make_sc_gather(_NPAD, _D)(xf, gidx)


def _combine_gather(y, slot):
    return _make_sc_gather(_N, _D)(y, slot)


def _comb_body(y0_ref, y1_ref, l_ref, m_ref, o_ref):
    eids = lax.broadcasted_iota(jnp.int32, (1, _E), 1)
    m = m_ref[...]
    l = l_ref[...]
    w0 = jnp.sum(jnp.where(m[:, 0:1] == eids, l, 0.0), axis=1, keepdims=True)
    w1 = jnp.sum(jnp.where(m[:, 1:2] == eids, l, 0.0), axis=1, keepdims=True)
    s = w0 + w1
    o_ref[...] = (y0_ref[...] * w0 + y1_ref[...] * w1) / s


def _combine(ys, logits, mapping):
    return pl.pallas_call(
        _comb_body,
        grid=(_T // _CT,),
        in_specs=[
            pl.BlockSpec((_CT, _D), lambda i: (i, 0)),
            pl.BlockSpec((_CT, _D), lambda i: (i + _T // _CT, 0)),
            pl.BlockSpec((_CT, _E), lambda i: (i, 0)),
            pl.BlockSpec((_CT, _TOPK), lambda i: (i, 0)),
        ],
        out_specs=pl.BlockSpec((_CT, _D), lambda i: (i, 0)),
        out_shape=jax.ShapeDtypeStruct((_T, _D), jnp.float32),
    )(ys, ys, logits, mapping)


def kernel(x, mapping, Wg, W1, b1, W2, b2):
    xf = x.reshape(_T, _D)
    logits = _gate(xf, Wg)

    mflat = jnp.concatenate([mapping[:, 0], mapping[:, 1]])
    oh = (mflat[:, None] == jnp.arange(_E, dtype=jnp.int32)[None, :])
    ranks = jnp.cumsum(oh.astype(jnp.int32), axis=0)
    rank = jnp.take_along_axis(ranks, mflat[:, None], axis=1)[:, 0] - 1
    counts = ranks[-1]
    padded = ((counts + _BLK - 1) // _BLK) * _BLK
    start = jnp.concatenate(
        [jnp.zeros((1,), padded.dtype), jnp.cumsum(padded)[:-1]])
    slot = (start[mflat] + rank).astype(jnp.int32)
    gidx = jnp.zeros((_NPAD,), jnp.int32).at[slot].set(
        jnp.arange(_N, dtype=jnp.int32) % _T)
    start_blk = (start // _BLK).astype(jnp.int32)
    bexp = (jnp.sum(
        jnp.arange(_NBLK, dtype=jnp.int32)[:, None] >= start_blk[None, :],
        axis=1) - 1).astype(jnp.int32)
    nused = ((jnp.sum(padded) // _BLK).astype(jnp.int32)).reshape(1)

    xs = _dispatch(xf, gidx)
    y = _ffn(bexp, nused, xs, W1, b1, W2, b2)
    ys = _combine_gather(y, slot)
    return _combine(ys, logits, mapping)


# speedup vs baseline: 1.3295x; 1.0391x over previous
import functools

import jax
import jax.numpy as jnp
from jax import lax
from jax.experimental import pallas as pl
from jax.experimental.pallas import tpu as pltpu
from jax.experimental.pallas import tpu_sc as plsc

_E = 8
_TOPK = 2
_D = 2048
_DFF = 8192
_T = 4096
_N = _T * _TOPK
_BLK = 256
_NBLK = _N // _BLK + _E
_NPAD = _NBLK * _BLK
_DFFT = 1024
_NJ = _DFF // _DFFT
_NC = 2
_NS = 16
_NW = _NC * _NS
_GCH = 16
_CT = 512


def _gate_body(x_ref, wg_ref, o_ref):
    logits = lax.dot_general(
        x_ref[...], wg_ref[...], (((1,), (1,)), ((), ())),
        preferred_element_type=jnp.float32)
    o_ref[...] = jax.nn.softmax(logits, axis=0)


def _gate(xf, wg):
    return pl.pallas_call(
        _gate_body,
        out_shape=jax.ShapeDtypeStruct((_T, _E), jnp.float32),
    )(xf, wg)


def _ffn_body(bexp_ref, nu_ref, x_ref, acc_ref, w1_ref, b1_ref, w2_ref,
              b2_ref, o_ref):
    del bexp_ref
    j = pl.program_id(0)
    i = pl.program_id(1)

    @pl.when(i < nu_ref[0])
    def _():
        h = jnp.dot(x_ref[...].astype(jnp.bfloat16),
                    w1_ref[0].astype(jnp.bfloat16),
                    preferred_element_type=jnp.float32)
        h = jax.nn.gelu(h + b1_ref[0])
        part = jnp.dot(h.astype(jnp.bfloat16),
                       w2_ref[0].astype(jnp.bfloat16),
                       preferred_element_type=jnp.float32)

        @pl.when(j == 0)
        def _():
            o_ref[...] = part + b2_ref[0]

        @pl.when(j > 0)
        def _():
            o_ref[...] = acc_ref[...] + part


def _ffn(bexp, nused, xs, W1, b1, W2, b2):
    grid_spec = pltpu.PrefetchScalarGridSpec(
        num_scalar_prefetch=2,
        grid=(_NJ, _NBLK),
        in_specs=[
            pl.BlockSpec((_BLK, _D), lambda j, i, be, nu: (i, 0)),
            pl.BlockSpec((_BLK, _D), lambda j, i, be, nu: (i, 0)),
            pl.BlockSpec((1, _D, _DFFT), lambda j, i, be, nu: (be[i], 0, j)),
            pl.BlockSpec((1, 1, _DFFT), lambda j, i, be, nu: (be[i], 0, j)),
            pl.BlockSpec((1, _DFFT, _D), lambda j, i, be, nu: (be[i], j, 0)),
            pl.BlockSpec((1, 1, _D), lambda j, i, be, nu: (be[i], 0, 0)),
        ],
        out_specs=pl.BlockSpec((_BLK, _D), lambda j, i, be, nu: (i, 0)),
    )
    acc = jnp.zeros((_NPAD, _D), jnp.float32)
    return pl.pallas_call(
        _ffn_body,
        grid_spec=grid_spec,
        out_shape=jax.ShapeDtypeStruct((_NPAD, _D), jnp.float32),
        input_output_aliases={3: 0},
        compiler_params=pltpu.CompilerParams(
            dimension_semantics=("arbitrary", "arbitrary"),
            vmem_limit_bytes=60 * 1024 * 1024),
    )(bexp, nused, xs, acc, W1, b1.reshape(_E, 1, _DFF), W2,
      b2.reshape(_E, 1, _D))


@functools.lru_cache(maxsize=None)
def _make_sc_gather(n_rows, width):
    per_w = n_rows // _NW
    nch = per_w // _GCH
    mesh = plsc.VectorSubcoreMesh(
        core_axis_name="c", subcore_axis_name="s",
        num_cores=_NC, num_subcores=_NS)

    assert nch % 2 == 0

    @functools.partial(
        pl.kernel,
        out_type=jax.ShapeDtypeStruct((n_rows, width), jnp.float32),
        mesh=mesh,
        scratch_types=[
            pltpu.VMEM((_GCH,), jnp.int32),
            pltpu.VMEM((_GCH,), jnp.int32),
            pltpu.VMEM((_GCH, width), jnp.float32),
            pltpu.VMEM((_GCH, width), jnp.float32),
            pltpu.SemaphoreType.DMA,
            pltpu.SemaphoreType.DMA,
        ],
    )
    def gather_k(table_hbm, idx_hbm, out_hbm, i0, i1, r0, r1, s0, s1):
        wid = lax.axis_index("s") * _NC + lax.axis_index("c")
        base = wid * per_w

        def fetch(g, idx_v, rows_v, sem):
            off = base + g * _GCH
            pltpu.sync_copy(idx_hbm.at[pl.ds(off, _GCH)], idx_v)
            pltpu.async_copy(table_hbm.at[idx_v], rows_v, sem)

        fetch(0, i0, r0, s0)

        def body(g0, carry):
            fetch(g0 + 1, i1, r1, s1)
            pltpu.make_async_copy(table_hbm.at[i0], r0, s0).wait()
            pltpu.sync_copy(r0, out_hbm.at[pl.ds(base + g0 * _GCH, _GCH)])

            @pl.when(g0 + 2 < nch)
            def _():
                fetch(g0 + 2, i0, r0, s0)

            pltpu.make_async_copy(table_hbm.at[i1], r1, s1).wait()
            pltpu.sync_copy(
                r1, out_hbm.at[pl.ds(base + (g0 + 1) * _GCH, _GCH)])
            return carry

        lax.fori_loop(0, nch // 2, lambda k, c: body(k * 2, c), 0)

    return gather_k


def _dispatch(xf, gidx):
    return _make_sc_gather(_NPAD, _D)(xf, gidx)


def _combine_gather(y, slot):
    return _make_sc_gather(_N, _D)(y, slot)


def _comb_body(y0_ref, y1_ref, l_ref, m_ref, o_ref):
    eids = lax.broadcasted_iota(jnp.int32, (1, _E), 1)
    m = m_ref[...]
    l = l_ref[...]
    w0 = jnp.sum(jnp.where(m[:, 0:1] == eids, l, 0.0), axis=1, keepdims=True)
    w1 = jnp.sum(jnp.where(m[:, 1:2] == eids, l, 0.0), axis=1, keepdims=True)
    s = w0 + w1
    o_ref[...] = (y0_ref[...] * w0 + y1_ref[...] * w1) / s


def _combine(ys, logits, mapping):
    return pl.pallas_call(
        _comb_body,
        grid=(_T // _CT,),
        in_specs=[
            pl.BlockSpec((_CT, _D), lambda i: (i, 0)),
            pl.BlockSpec((_CT, _D), lambda i: (i + _T // _CT, 0)),
            pl.BlockSpec((_CT, _E), lambda i: (i, 0)),
            pl.BlockSpec((_CT, _TOPK), lambda i: (i, 0)),
        ],
        out_specs=pl.BlockSpec((_CT, _D), lambda i: (i, 0)),
        out_shape=jax.ShapeDtypeStruct((_T, _D), jnp.float32),
    )(ys, ys, logits, mapping)


def kernel(x, mapping, Wg, W1, b1, W2, b2):
    xf = x.reshape(_T, _D)
    logits = _gate(xf, Wg)

    mflat = jnp.concatenate([mapping[:, 0], mapping[:, 1]])
    dup = mapping[:, 1] == mapping[:, 0]
    active = jnp.concatenate([jnp.ones((_T,), jnp.bool_), ~dup])
    oh = (mflat[:, None] == jnp.arange(_E, dtype=jnp.int32)[None, :])
    oh = oh & active[:, None]
    ranks = jnp.cumsum(oh.astype(jnp.int32), axis=0)
    rank = jnp.take_along_axis(ranks, mflat[:, None], axis=1)[:, 0] - 1
    counts = ranks[-1]
    padded = ((counts + _BLK - 1) // _BLK) * _BLK
    start = jnp.concatenate(
        [jnp.zeros((1,), padded.dtype), jnp.cumsum(padded)[:-1]])
    slot = (start[mflat] + rank).astype(jnp.int32)
    slot = jnp.where(active, slot,
                     jnp.concatenate([slot[:_T], slot[:_T]]))
    gidx = jnp.zeros((_NPAD,), jnp.int32).at[
        jnp.where(active, slot, _NPAD)].set(
        jnp.arange(_N, dtype=jnp.int32) % _T, mode="drop")
    start_blk = (start // _BLK).astype(jnp.int32)
    bexp = (jnp.sum(
        jnp.arange(_NBLK, dtype=jnp.int32)[:, None] >= start_blk[None, :],
        axis=1) - 1).astype(jnp.int32)
    nused = ((jnp.sum(padded) // _BLK).astype(jnp.int32)).reshape(1)

    xs = _dispatch(xf, gidx)
    y = _ffn(bexp, nused, xs, W1, b1, W2, b2)
    ys = _combine_gather(y, slot)
    return _combine(ys, logits, mapping)


# speedup vs baseline: 1.3481x; 1.0140x over previous
import functools

import jax
import jax.numpy as jnp
from jax import lax
from jax.experimental import pallas as pl
from jax.experimental.pallas import tpu as pltpu
from jax.experimental.pallas import tpu_sc as plsc

_E = 8
_TOPK = 2
_D = 2048
_DFF = 8192
_T = 4096
_N = _T * _TOPK
_BLK = 256
_NBLK = _N // _BLK + _E
_NPAD = _NBLK * _BLK
_DFFT = 1024
_NJ = _DFF // _DFFT
_NC = 2
_NS = 16
_NW = _NC * _NS
_GCH = 16
_CT = 512


def _gate_body(x_ref, wg_ref, o_ref):
    logits = lax.dot_general(
        x_ref[...], wg_ref[...], (((1,), (1,)), ((), ())),
        preferred_element_type=jnp.float32)
    o_ref[...] = jax.nn.softmax(logits, axis=0)


def _gate(xf, wg):
    return pl.pallas_call(
        _gate_body,
        out_shape=jax.ShapeDtypeStruct((_T, _E), jnp.float32),
    )(xf, wg)


def _ffn_part(bexp, nused, xs_part, acc, W1, b1, W2, b2, off, nblk):
    grid_spec = pltpu.PrefetchScalarGridSpec(
        num_scalar_prefetch=2,
        grid=(_NJ, nblk),
        in_specs=[
            pl.BlockSpec((_BLK, _D), lambda j, i, be, nu: (i, 0)),
            pl.BlockSpec((_BLK, _D), lambda j, i, be, nu: (i + off, 0)),
            pl.BlockSpec((1, _D, _DFFT),
                         lambda j, i, be, nu: (be[i + off], 0, j)),
            pl.BlockSpec((1, 1, _DFFT),
                         lambda j, i, be, nu: (be[i + off], 0, j)),
            pl.BlockSpec((1, _DFFT, _D),
                         lambda j, i, be, nu: (be[i + off], j, 0)),
            pl.BlockSpec((1, 1, _D),
                         lambda j, i, be, nu: (be[i + off], 0, 0)),
        ],
        out_specs=pl.BlockSpec((_BLK, _D), lambda j, i, be, nu: (i + off, 0)),
    )

    def body(bexp_ref, nu_ref, x_ref, acc_ref, w1_ref, b1_ref, w2_ref,
             b2_ref, o_ref):
        del bexp_ref
        j = pl.program_id(0)
        i = pl.program_id(1)

        @pl.when(i + off < nu_ref[0])
        def _():
            h = jnp.dot(x_ref[...].astype(jnp.bfloat16),
                        w1_ref[0].astype(jnp.bfloat16),
                        preferred_element_type=jnp.float32)
            h = jax.nn.gelu(h + b1_ref[0])
            part = jnp.dot(h.astype(jnp.bfloat16),
                           w2_ref[0].astype(jnp.bfloat16),
                           preferred_element_type=jnp.float32)

            @pl.when(j == 0)
            def _():
                o_ref[...] = part + b2_ref[0]

            @pl.when(j > 0)
            def _():
                o_ref[...] = acc_ref[...] + part

    return pl.pallas_call(
        body,
        grid_spec=grid_spec,
        out_shape=jax.ShapeDtypeStruct((_NPAD, _D), jnp.float32),
        input_output_aliases={3: 0},
        compiler_params=pltpu.CompilerParams(
            dimension_semantics=("arbitrary", "arbitrary"),
            vmem_limit_bytes=60 * 1024 * 1024),
    )(bexp, nused, xs_part, acc, W1, b1.reshape(_E, 1, _DFF), W2,
      b2.reshape(_E, 1, _D))


@functools.lru_cache(maxsize=None)
def _make_sc_gather(n_rows, width):
    per_w = n_rows // _NW
    nch = per_w // _GCH
    mesh = plsc.VectorSubcoreMesh(
        core_axis_name="c", subcore_axis_name="s",
        num_cores=_NC, num_subcores=_NS)

    assert nch % 2 == 0

    @functools.partial(
        pl.kernel,
        out_type=jax.ShapeDtypeStruct((n_rows, width), jnp.float32),
        mesh=mesh,
        scratch_types=[
            pltpu.VMEM((_GCH,), jnp.int32),
            pltpu.VMEM((_GCH,), jnp.int32),
            pltpu.VMEM((_GCH, width), jnp.float32),
            pltpu.VMEM((_GCH, width), jnp.float32),
            pltpu.SemaphoreType.DMA,
            pltpu.SemaphoreType.DMA,
        ],
    )
    def gather_k(table_hbm, idx_hbm, out_hbm, i0, i1, r0, r1, s0, s1):
        wid = lax.axis_index("s") * _NC + lax.axis_index("c")
        base = wid * per_w

        def fetch(g, idx_v, rows_v, sem):
            off = base + g * _GCH
            pltpu.sync_copy(idx_hbm.at[pl.ds(off, _GCH)], idx_v)
            pltpu.async_copy(table_hbm.at[idx_v], rows_v, sem)

        fetch(0, i0, r0, s0)

        def body(g0, carry):
            fetch(g0 + 1, i1, r1, s1)
            pltpu.make_async_copy(table_hbm.at[i0], r0, s0).wait()
            pltpu.sync_copy(r0, out_hbm.at[pl.ds(base + g0 * _GCH, _GCH)])

            @pl.when(g0 + 2 < nch)
            def _():
                fetch(g0 + 2, i0, r0, s0)

            pltpu.make_async_copy(table_hbm.at[i1], r1, s1).wait()
            pltpu.sync_copy(
                r1, out_hbm.at[pl.ds(base + (g0 + 1) * _GCH, _GCH)])
            return carry

        lax.fori_loop(0, nch // 2, lambda k, c: body(k * 2, c), 0)

    return gather_k


def _combine_gather(y, slot):
    return _make_sc_gather(_N, _D)(y, slot)


def _comb_body(y0_ref, y1_ref, l_ref, m_ref, o_ref):
    eids = lax.broadcasted_iota(jnp.int32, (1, _E), 1)
    m = m_ref[...]
    l = l_ref[...]
    w0 = jnp.sum(jnp.where(m[:, 0:1] == eids, l, 0.0), axis=1, keepdims=True)
    w1 = jnp.sum(jnp.where(m[:, 1:2] == eids, l, 0.0), axis=1, keepdims=True)
    s = w0 + w1
    o_ref[...] = (y0_ref[...] * w0 + y1_ref[...] * w1) / s


def _combine(ys, logits, mapping):
    return pl.pallas_call(
        _comb_body,
        grid=(_T // _CT,),
        in_specs=[
            pl.BlockSpec((_CT, _D), lambda i: (i, 0)),
            pl.BlockSpec((_CT, _D), lambda i: (i + _T // _CT, 0)),
            pl.BlockSpec((_CT, _E), lambda i: (i, 0)),
            pl.BlockSpec((_CT, _TOPK), lambda i: (i, 0)),
        ],
        out_specs=pl.BlockSpec((_CT, _D), lambda i: (i, 0)),
        out_shape=jax.ShapeDtypeStruct((_T, _D), jnp.float32),
    )(ys, ys, logits, mapping)


def kernel(x, mapping, Wg, W1, b1, W2, b2):
    xf = x.reshape(_T, _D)
    logits = _gate(xf, Wg)

    mflat = jnp.concatenate([mapping[:, 0], mapping[:, 1]])
    dup = mapping[:, 1] == mapping[:, 0]
    active = jnp.concatenate([jnp.ones((_T,), jnp.bool_), ~dup])
    oh = (mflat[:, None] == jnp.arange(_E, dtype=jnp.int32)[None, :])
    oh = oh & active[:, None]
    ranks = jnp.cumsum(oh.astype(jnp.int32), axis=0)
    rank = jnp.take_along_axis(ranks, mflat[:, None], axis=1)[:, 0] - 1
    counts = ranks[-1]
    padded = ((counts + _BLK - 1) // _BLK) * _BLK
    start = jnp.concatenate(
        [jnp.zeros((1,), padded.dtype), jnp.cumsum(padded)[:-1]])
    slot = (start[mflat] + rank).astype(jnp.int32)
    slot = jnp.where(active, slot,
                     jnp.concatenate([slot[:_T], slot[:_T]]))
    gidx = jnp.zeros((_NPAD,), jnp.int32).at[
        jnp.where(active, slot, _NPAD)].set(
        jnp.arange(_N, dtype=jnp.int32) % _T, mode="drop")
    start_blk = (start // _BLK).astype(jnp.int32)
    bexp = (jnp.sum(
        jnp.arange(_NBLK, dtype=jnp.int32)[:, None] >= start_blk[None, :],
        axis=1) - 1).astype(jnp.int32)
    nused = ((jnp.sum(padded) // _BLK).astype(jnp.int32)).reshape(1)

    half_rows = _NPAD // 2
    half_blk = _NBLK // 2
    xs_a = _make_sc_gather(half_rows, _D)(xf, gidx[:half_rows])
    xs_b = _make_sc_gather(half_rows, _D)(xf, gidx[half_rows:])
    acc = jnp.zeros((_NPAD, _D), jnp.float32)
    y1 = _ffn_part(bexp, nused, xs_a, acc, W1, b1, W2, b2, 0, half_blk)
    y = _ffn_part(bexp, nused, xs_b, y1, W1, b1, W2, b2, half_blk, half_blk)
    ys = _combine_gather(y, slot)
    return _combine(ys, logits, mapping)


# speedup vs baseline: 1.4410x; 1.0689x over previous
import functools

import jax
import jax.numpy as jnp
from jax import lax
from jax.experimental import pallas as pl
from jax.experimental.pallas import tpu as pltpu
from jax.experimental.pallas import tpu_sc as plsc

_E = 8
_TOPK = 2
_D = 2048
_DFF = 8192
_T = 4096
_N = _T * _TOPK
_BLK = 256
_NBLK = _N // _BLK + _E
_NPAD = _NBLK * _BLK
_DFFT = 1024
_NJ = _DFF // _DFFT
_NC = 2
_NS = 16
_NW = _NC * _NS
_GCH = 16
_CT = 512


def _gate_body(x_ref, wg_ref, o_ref):
    logits = lax.dot_general(
        x_ref[...], wg_ref[...], (((1,), (1,)), ((), ())),
        preferred_element_type=jnp.float32)
    o_ref[...] = jax.nn.softmax(logits, axis=0)


def _gate(xf, wg):
    return pl.pallas_call(
        _gate_body,
        out_shape=jax.ShapeDtypeStruct((_T, _E), jnp.float32),
    )(xf, wg)


def _ffn_part(bexp, nused, xs_part, acc, W1, b1, W2, b2, off, nblk):
    def _clamp(i, nu):
        return jnp.clip(jnp.minimum(i + off, nu[0] - 1), off, off + nblk - 1)

    in_specs = [
        pl.BlockSpec((_BLK, _D),
                     lambda j, i, be, nu: (_clamp(i, nu) - off, 0)),
        pl.BlockSpec((_BLK, _D), lambda j, i, be, nu: (_clamp(i, nu), 0)),
        pl.BlockSpec((1, _D, _DFFT),
                     lambda j, i, be, nu: (be[_clamp(i, nu)], 0, j)),
        pl.BlockSpec((1, 1, _DFFT),
                     lambda j, i, be, nu: (be[_clamp(i, nu)], 0, j)),
        pl.BlockSpec((1, _DFFT, _D),
                     lambda j, i, be, nu: (be[_clamp(i, nu)], j, 0)),
        pl.BlockSpec((1, 1, _D),
                     lambda j, i, be, nu: (be[_clamp(i, nu)], 0, 0)),
    ]
    grid_spec = pltpu.PrefetchScalarGridSpec(
        num_scalar_prefetch=2,
        grid=(_NJ, nblk),
        in_specs=in_specs,
        out_specs=pl.BlockSpec((_BLK, _D),
                               lambda j, i, be, nu: (_clamp(i, nu), 0)),
    )

    def body(bexp_ref, nu_ref, x_ref, acc_ref, w1_ref, b1_ref, w2_ref,
             b2_ref, o_ref):
        del bexp_ref
        j = pl.program_id(0)
        i = pl.program_id(1)

        @pl.when(i + off < nu_ref[0])
        def _():
            h = jnp.dot(x_ref[...].astype(jnp.bfloat16),
                        w1_ref[0].astype(jnp.bfloat16),
                        preferred_element_type=jnp.float32)
            h = jax.nn.gelu(h + b1_ref[0])
            part = jnp.dot(h.astype(jnp.bfloat16),
                           w2_ref[0].astype(jnp.bfloat16),
                           preferred_element_type=jnp.float32)

            @pl.when(j == 0)
            def _():
                o_ref[...] = part + b2_ref[0]

            @pl.when(j > 0)
            def _():
                o_ref[...] = acc_ref[...] + part

    return pl.pallas_call(
        body,
        grid_spec=grid_spec,
        out_shape=jax.ShapeDtypeStruct((_NPAD, _D), jnp.float32),
        input_output_aliases={3: 0},
        compiler_params=pltpu.CompilerParams(
            dimension_semantics=("arbitrary", "arbitrary"),
            vmem_limit_bytes=60 * 1024 * 1024),
    )(bexp, nused, xs_part, acc, W1, b1.reshape(_E, 1, _DFF), W2,
      b2.reshape(_E, 1, _D))


@functools.lru_cache(maxsize=None)
def _make_sc_gather(n_rows, width, dtype=jnp.float32):
    per_w = n_rows // _NW
    nch = per_w // _GCH
    mesh = plsc.VectorSubcoreMesh(
        core_axis_name="c", subcore_axis_name="s",
        num_cores=_NC, num_subcores=_NS)

    assert nch % 2 == 0

    @functools.partial(
        pl.kernel,
        out_type=jax.ShapeDtypeStruct((n_rows, width), dtype),
        mesh=mesh,
        scratch_types=[
            pltpu.VMEM((_GCH,), jnp.int32),
            pltpu.VMEM((_GCH,), jnp.int32),
            pltpu.VMEM((_GCH, width), dtype),
            pltpu.VMEM((_GCH, width), dtype),
            pltpu.SemaphoreType.DMA,
            pltpu.SemaphoreType.DMA,
        ],
    )
    def gather_k(table_hbm, idx_hbm, out_hbm, i0, i1, r0, r1, s0, s1):
        wid = lax.axis_index("s") * _NC + lax.axis_index("c")
        base = wid * per_w

        def fetch(g, idx_v, rows_v, sem):
            off = base + g * _GCH
            pltpu.sync_copy(idx_hbm.at[pl.ds(off, _GCH)], idx_v)
            pltpu.async_copy(table_hbm.at[idx_v], rows_v, sem)

        fetch(0, i0, r0, s0)

        def body(g0, carry):
            fetch(g0 + 1, i1, r1, s1)
            pltpu.make_async_copy(table_hbm.at[i0], r0, s0).wait()
            pltpu.sync_copy(r0, out_hbm.at[pl.ds(base + g0 * _GCH, _GCH)])

            @pl.when(g0 + 2 < nch)
            def _():
                fetch(g0 + 2, i0, r0, s0)

            pltpu.make_async_copy(table_hbm.at[i1], r1, s1).wait()
            pltpu.sync_copy(
                r1, out_hbm.at[pl.ds(base + (g0 + 1) * _GCH, _GCH)])
            return carry

        lax.fori_loop(0, nch // 2, lambda k, c: body(k * 2, c), 0)

    return gather_k


def _combine_gather(y, slot):
    return _make_sc_gather(_N, _D)(y, slot)


def _comb_body(y0_ref, y1_ref, l_ref, m_ref, o_ref):
    eids = lax.broadcasted_iota(jnp.int32, (1, _E), 1)
    m = m_ref[...]
    l = l_ref[...]
    w0 = jnp.sum(jnp.where(m[:, 0:1] == eids, l, 0.0), axis=1, keepdims=True)
    w1 = jnp.sum(jnp.where(m[:, 1:2] == eids, l, 0.0), axis=1, keepdims=True)
    s = w0 + w1
    o_ref[...] = (y0_ref[...] * w0 + y1_ref[...] * w1) / s


def _combine(ys, logits, mapping):
    return pl.pallas_call(
        _comb_body,
        grid=(_T // _CT,),
        in_specs=[
            pl.BlockSpec((_CT, _D), lambda i: (i, 0)),
            pl.BlockSpec((_CT, _D), lambda i: (i + _T // _CT, 0)),
            pl.BlockSpec((_CT, _E), lambda i: (i, 0)),
            pl.BlockSpec((_CT, _TOPK), lambda i: (i, 0)),
        ],
        out_specs=pl.BlockSpec((_CT, _D), lambda i: (i, 0)),
        out_shape=jax.ShapeDtypeStruct((_T, _D), jnp.float32),
    )(ys, ys, logits, mapping)


def kernel(x, mapping, Wg, W1, b1, W2, b2):
    xf = x.reshape(_T, _D)
    logits = _gate(xf, Wg)

    mflat = jnp.concatenate([mapping[:, 0], mapping[:, 1]])
    dup = mapping[:, 1] == mapping[:, 0]
    active = jnp.concatenate([jnp.ones((_T,), jnp.bool_), ~dup])
    oh = (mflat[:, None] == jnp.arange(_E, dtype=jnp.int32)[None, :])
    oh = oh & active[:, None]
    ranks = jnp.cumsum(oh.astype(jnp.int32), axis=0)
    rank = jnp.take_along_axis(ranks, mflat[:, None], axis=1)[:, 0] - 1
    counts = ranks[-1]
    padded = ((counts + _BLK - 1) // _BLK) * _BLK
    start = jnp.concatenate(
        [jnp.zeros((1,), padded.dtype), jnp.cumsum(padded)[:-1]])
    slot = (start[mflat] + rank).astype(jnp.int32)
    slot = jnp.where(active, slot,
                     jnp.concatenate([slot[:_T], slot[:_T]]))
    gidx = jnp.zeros((_NPAD,), jnp.int32).at[
        jnp.where(active, slot, _NPAD)].set(
        jnp.arange(_N, dtype=jnp.int32) % _T, mode="drop")
    start_blk = (start // _BLK).astype(jnp.int32)
    bexp = (jnp.sum(
        jnp.arange(_NBLK, dtype=jnp.int32)[:, None] >= start_blk[None, :],
        axis=1) - 1).astype(jnp.int32)
    nused = ((jnp.sum(padded) // _BLK).astype(jnp.int32)).reshape(1)

    half_rows = _NPAD // 2
    half_blk = _NBLK // 2
    xs_a = _make_sc_gather(half_rows, _D)(xf, gidx[:half_rows])
    xs_b = _make_sc_gather(half_rows, _D)(xf, gidx[half_rows:])
    acc = jnp.zeros((_NPAD, _D), jnp.float32)
    y1 = _ffn_part(bexp, nused, xs_a, acc, W1, b1, W2, b2, 0, half_blk)
    y = _ffn_part(bexp, nused, xs_b, y1, W1, b1, W2, b2, half_blk, half_blk)
    ys = _combine_gather(y, slot)
    return _combine(ys, logits, mapping)
